# baseline (speedup 1.0000x reference)
import jax
import jax.numpy as jnp
from jax.experimental import pallas as pl
from jax.experimental.pallas import tpu as pltpu

BATCH = 16
N = 20000
PRE = 6000
POST = 300
IOU_THR = 0.7
OUT_PAD = 384
PADN = 20480
BLKS = 160
OROWS = 48
CW = OROWS * 128


def _monotone_key(scores):
    i = jax.lax.bitcast_convert_type(scores, jnp.int32)
    return jnp.where(i < 0, i ^ jnp.int32(0x7FFFFFFF), i)


def _nms_kernel(scores_ref, deltas_ref, anchors_ref, out_ref, s_ref):
    scores = scores_ref[...]
    key = _monotone_key(scores)
    lane = jax.lax.broadcasted_iota(jnp.int32, (BATCH, N), 1)

    def count_ge(thr):
        return jnp.sum((key >= thr).astype(jnp.int32), axis=1, keepdims=True)

    big = jnp.full((BATCH, 1), jnp.int32(-2147483648))
    zero = jnp.zeros((BATCH, 1), jnp.int32)
    cur = jnp.where(count_ge(zero) >= PRE, zero, big)

    def bis_body(k, cur):
        bit = jnp.int32(1) << (jnp.int32(30) - k)
        cand = cur | bit
        return jnp.where(count_ge(cand) >= PRE, cand, cur)

    thr = jax.lax.fori_loop(0, 31, bis_body, cur)

    gt = jnp.sum((key > thr).astype(jnp.int32), axis=1, keepdims=True)
    need = PRE - gt
    eq = key == thr

    def idx_body(k, cur):
        bit = jnp.int32(1) << (jnp.int32(14) - k)
        cand = cur | bit
        cnt = jnp.sum((eq & (lane < cand)).astype(jnp.int32), axis=1,
                      keepdims=True)
        return jnp.where(cnt < need, cand, cur)

    idx_thr = jax.lax.fori_loop(0, 15, idx_body,
                                jnp.zeros((BATCH, 1), jnp.int32))
    valid = (key > thr) | (eq & (lane <= idx_thr))

    a_y1 = anchors_ref[0:1, :]
    a_x1 = anchors_ref[1:2, :]
    a_y2 = anchors_ref[2:3, :]
    a_x2 = anchors_ref[3:4, :]
    anc_w = a_x2 - a_x1
    anc_h = a_y2 - a_y1
    anc_cx = a_x1 + 0.5 * anc_w
    anc_cy = a_y1 + 0.5 * anc_h
    d_y = deltas_ref[0] * jnp.float32(0.1)
    d_x = deltas_ref[1] * jnp.float32(0.1)
    d_h = deltas_ref[2] * jnp.float32(0.2)
    d_w = deltas_ref[3] * jnp.float32(0.2)
    bb_w = jnp.exp(d_w) * anc_w
    bb_h = jnp.exp(d_h) * anc_h
    bb_cx = d_x * anc_w + anc_cx
    bb_cy = d_y * anc_h + anc_cy
    y1 = bb_cy - 0.5 * bb_h
    x1 = bb_cx - 0.5 * bb_w
    y2 = bb_h + y1
    x2 = bb_w + x1

    zf = jnp.float32(0.0)
    pad = jnp.zeros((BATCH, PADN - N), jnp.float32)
    vf = jnp.concatenate([jnp.where(valid, jnp.float32(1.0), zf), pad],
                         axis=1)
    V = vf.reshape(BATCH * BLKS, 128)
    li = jax.lax.broadcasted_iota(jnp.int32, (128, 128), 0)
    lj = jax.lax.broadcasted_iota(jnp.int32, (128, 128), 1)
    T128 = jnp.where(li < lj, jnp.float32(1.0), zf)
    rank_ex = jax.lax.dot_general(
        V, T128, (((1,), (0,)), ((), ())),
        preferred_element_type=jnp.float32,
        precision=jax.lax.Precision.HIGHEST)
    kidx_f = jax.lax.broadcasted_iota(
        jnp.int32, (BATCH * BLKS, 128), 1).astype(jnp.float32)
    lo = jnp.zeros((BATCH * BLKS, 128), jnp.int32)
    for bit in (64, 32, 16, 8, 4, 2, 1):
        c = lo + bit
        rc = jnp.take_along_axis(rank_ex, c, axis=1)
        lo = jnp.where(rc <= kidx_f, c, lo)
    cnt = rank_ex[:, 127:128] + V[:, 127:128]
    keep = kidx_f < cnt

    C = jnp.sum(vf.reshape(BATCH, BLKS, 128), axis=2)
    i160 = jax.lax.broadcasted_iota(jnp.int32, (BLKS, BLKS), 0)
    j160 = jax.lax.broadcasted_iota(jnp.int32, (BLKS, BLKS), 1)
    T160 = jnp.where(i160 < j160, jnp.float32(1.0), zf)
    O = jax.lax.dot_general(
        C, T160, (((1,), (0,)), ((), ())),
        preferred_element_type=jnp.float32,
        precision=jax.lax.Precision.HIGHEST)
    Oi = O.astype(jnp.int32)
    shift = jnp.broadcast_to((Oi & 127)[:, :, None],
                             (BATCH, BLKS, 128)).reshape(BATCH * BLKS, 128)
    lane128 = jax.lax.broadcasted_iota(jnp.int32, (BATCH * BLKS, 128), 1)
    idxrot = (lane128 - shift + 128) & 127
    real = idxrot.astype(jnp.float32) < cnt
    partA = real & (lane128 >= shift)
    partB = real & (lane128 < shift)
    m0 = Oi >> 7
    mm = jax.lax.broadcasted_iota(jnp.int32, (BATCH, BLKS, OROWS), 2)
    RA = jnp.where(m0[:, :, None] == mm, jnp.float32(1.0), zf)
    RB = jnp.where((m0 + 1)[:, :, None] == mm, jnp.float32(1.0), zf)
    glo = jnp.take_along_axis(lo, idxrot, axis=1)

    def compact(p):
        pp = jnp.concatenate([p, pad], axis=1).reshape(BATCH * BLKS, 128)
        rot = jnp.take_along_axis(pp, glo, axis=1)
        mA = jnp.where(partA, rot, zf).reshape(BATCH, BLKS, 128)
        mB = jnp.where(partB, rot, zf).reshape(BATCH, BLKS, 128)
        o3 = (jax.lax.dot_general(
                  RA, mA, (((1,), (1,)), ((0,), (0,))),
                  preferred_element_type=jnp.float32,
                  precision=jax.lax.Precision.HIGHEST)
              + jax.lax.dot_general(
                  RB, mB, (((1,), (1,)), ((0,), (0,))),
                  preferred_element_type=jnp.float32,
                  precision=jax.lax.Precision.HIGHEST))
        return o3.reshape(BATCH, CW)

    clane = jax.lax.broadcasted_iota(jnp.int32, (BATCH, CW), 1)
    yy1 = compact(y1)
    xx1 = compact(x1)
    yy2 = compact(y2)
    xx2 = compact(x2)
    s_c = jnp.where(clane < PRE, compact(scores), jnp.float32(-1.0))
    ab = (jnp.float32(IOU_THR)
          * jnp.maximum(yy2 - yy1, 0.0) * jnp.maximum(xx2 - xx1, 0.0))

    s_ref[...] = s_c
    out_ref[...] = jnp.zeros((4, BATCH, OUT_PAD), jnp.float32)
    out_lane = jax.lax.broadcasted_iota(jnp.int32, (BATCH, OUT_PAD), 1)
    lane = clane

    def one_pick(s):
        m = jnp.max(s, axis=1, keepdims=True)
        anyv = m >= 0.0
        pick = (s == m) & anyv
        pos = jnp.min(jnp.where(pick, lane, jnp.int32(CW)), axis=1,
                      keepdims=True)
        onehot = lane == pos
        oh_f = jnp.where(onehot, jnp.float32(1.0), jnp.float32(0.0))

        def sel(plane):
            return jnp.sum(plane * oh_f, axis=1, keepdims=True)

        by1 = sel(yy1)
        bx1 = sel(xx1)
        by2 = sel(yy2)
        bx2 = sel(xx2)

        inter = (jnp.maximum(jnp.minimum(by2, yy2) - jnp.maximum(by1, yy1),
                             0.0)
                 * jnp.maximum(jnp.minimum(bx2, xx2) - jnp.maximum(bx1, xx1),
                               0.0))
        area_a = jnp.maximum(by2 - by1, 0.0) * jnp.maximum(bx2 - bx1, 0.0)
        supp = (jnp.float32(1.0 + IOU_THR) * inter
                > jnp.float32(IOU_THR) * area_a + ab)
        supp = supp | onehot
        s_next = jnp.where(anyv & supp, jnp.float32(-1.0), s)
        return s_next, (by1, bx1, by2, bx2), anyv

    def write_out(o, i, box, anyv):
        wmask = (out_lane == i) & anyv
        return [jnp.where(wmask, jnp.clip(b, 0.0, 1.0), oo)
                for b, oo in zip(box, o)]

    def body(i, _):
        s = s_ref[...]
        o = [out_ref[0], out_ref[1], out_ref[2], out_ref[3]]
        for k in range(6):
            s, box, anyv = one_pick(s)
            o = write_out(o, 6 * i + k, box, anyv)
        s_ref[...] = s
        out_ref[...] = jnp.stack(o, axis=0)
        return 0

    jax.lax.fori_loop(0, POST // 6, body, 0)


@jax.jit
def kernel(rpn_bbox_deltas, rpn_labels, anchors):
    deltas_t = jnp.transpose(rpn_bbox_deltas, (2, 0, 1))
    anchors_t = jnp.transpose(anchors, (1, 0))
    out = pl.pallas_call(
        _nms_kernel,
        out_shape=jax.ShapeDtypeStruct((4, BATCH, OUT_PAD), jnp.float32),
        scratch_shapes=[pltpu.VMEM((BATCH, CW), jnp.float32)],
    )(rpn_labels, deltas_t, anchors_t)
    return jnp.transpose(out[:, :, :POST], (1, 2, 0))


# speedup vs baseline: 1.5549x; 1.5549x over previous
import jax
import jax.numpy as jnp
from jax.experimental import pallas as pl
from jax.experimental.pallas import tpu as pltpu

BATCH = 16
N = 20000
PRE = 6000
POST = 300
IOU_THR = 0.7
OUT_PAD = 384
PADN = 20480
BLKS = 160
OROWS = 48
CW = OROWS * 128


def _monotone_key(scores):
    i = jax.lax.bitcast_convert_type(scores, jnp.int32)
    return jnp.where(i < 0, i ^ jnp.int32(0x7FFFFFFF), i)


def _nms_kernel(scores_ref, deltas_ref, anchors_ref, out_ref):
    scores = scores_ref[...]
    key = _monotone_key(scores)
    lane = jax.lax.broadcasted_iota(jnp.int32, (BATCH, N), 1)

    def count_ge(thr):
        return jnp.sum((key >= thr).astype(jnp.int32), axis=1, keepdims=True)

    big = jnp.full((BATCH, 1), jnp.int32(-2147483648))
    zero = jnp.zeros((BATCH, 1), jnp.int32)
    cur = jnp.where(count_ge(zero) >= PRE, zero, big)

    def bis_body(k, cur):
        bit = jnp.int32(1) << (jnp.int32(30) - k)
        cand = cur | bit
        return jnp.where(count_ge(cand) >= PRE, cand, cur)

    thr = jax.lax.fori_loop(0, 31, bis_body, cur)

    gt = jnp.sum((key > thr).astype(jnp.int32), axis=1, keepdims=True)
    need = PRE - gt
    eq = key == thr

    def idx_body(k, cur):
        bit = jnp.int32(1) << (jnp.int32(14) - k)
        cand = cur | bit
        cnt = jnp.sum((eq & (lane < cand)).astype(jnp.int32), axis=1,
                      keepdims=True)
        return jnp.where(cnt < need, cand, cur)

    idx_thr = jax.lax.fori_loop(0, 15, idx_body,
                                jnp.zeros((BATCH, 1), jnp.int32))
    valid = (key > thr) | (eq & (lane <= idx_thr))

    a_y1 = anchors_ref[0:1, :]
    a_x1 = anchors_ref[1:2, :]
    a_y2 = anchors_ref[2:3, :]
    a_x2 = anchors_ref[3:4, :]
    anc_w = a_x2 - a_x1
    anc_h = a_y2 - a_y1
    anc_cx = a_x1 + 0.5 * anc_w
    anc_cy = a_y1 + 0.5 * anc_h
    d_y = deltas_ref[0] * jnp.float32(0.1)
    d_x = deltas_ref[1] * jnp.float32(0.1)
    d_h = deltas_ref[2] * jnp.float32(0.2)
    d_w = deltas_ref[3] * jnp.float32(0.2)
    bb_w = jnp.exp(d_w) * anc_w
    bb_h = jnp.exp(d_h) * anc_h
    bb_cx = d_x * anc_w + anc_cx
    bb_cy = d_y * anc_h + anc_cy
    y1 = bb_cy - 0.5 * bb_h
    x1 = bb_cx - 0.5 * bb_w
    y2 = bb_h + y1
    x2 = bb_w + x1

    zf = jnp.float32(0.0)
    pad = jnp.zeros((BATCH, PADN - N), jnp.float32)
    vf = jnp.concatenate([jnp.where(valid, jnp.float32(1.0), zf), pad],
                         axis=1)
    V = vf.reshape(BATCH * BLKS, 128)
    li = jax.lax.broadcasted_iota(jnp.int32, (128, 128), 0)
    lj = jax.lax.broadcasted_iota(jnp.int32, (128, 128), 1)
    T128 = jnp.where(li < lj, jnp.float32(1.0), zf)
    rank_ex = jax.lax.dot_general(
        V, T128, (((1,), (0,)), ((), ())),
        preferred_element_type=jnp.float32,
        precision=jax.lax.Precision.HIGHEST)
    kidx_f = jax.lax.broadcasted_iota(
        jnp.int32, (BATCH * BLKS, 128), 1).astype(jnp.float32)
    lo = jnp.zeros((BATCH * BLKS, 128), jnp.int32)
    for bit in (64, 32, 16, 8, 4, 2, 1):
        c = lo + bit
        rc = jnp.take_along_axis(rank_ex, c, axis=1)
        lo = jnp.where(rc <= kidx_f, c, lo)
    cnt = rank_ex[:, 127:128] + V[:, 127:128]
    keep = kidx_f < cnt

    C = jnp.sum(vf.reshape(BATCH, BLKS, 128), axis=2)
    i160 = jax.lax.broadcasted_iota(jnp.int32, (BLKS, BLKS), 0)
    j160 = jax.lax.broadcasted_iota(jnp.int32, (BLKS, BLKS), 1)
    T160 = jnp.where(i160 < j160, jnp.float32(1.0), zf)
    O = jax.lax.dot_general(
        C, T160, (((1,), (0,)), ((), ())),
        preferred_element_type=jnp.float32,
        precision=jax.lax.Precision.HIGHEST)
    Oi = O.astype(jnp.int32)
    shift = jnp.broadcast_to((Oi & 127)[:, :, None],
                             (BATCH, BLKS, 128)).reshape(BATCH * BLKS, 128)
    lane128 = jax.lax.broadcasted_iota(jnp.int32, (BATCH * BLKS, 128), 1)
    idxrot = (lane128 - shift + 128) & 127
    real = idxrot.astype(jnp.float32) < cnt
    partA = real & (lane128 >= shift)
    partB = real & (lane128 < shift)
    m0 = Oi >> 7
    mm = jax.lax.broadcasted_iota(jnp.int32, (BATCH, BLKS, OROWS), 2)
    RA = jnp.where(m0[:, :, None] == mm, jnp.float32(1.0), zf)
    RB = jnp.where((m0 + 1)[:, :, None] == mm, jnp.float32(1.0), zf)
    glo = jnp.take_along_axis(lo, idxrot, axis=1)

    def compact(p):
        pp = jnp.concatenate([p, pad], axis=1).reshape(BATCH * BLKS, 128)
        rot = jnp.take_along_axis(pp, glo, axis=1)
        mA = jnp.where(partA, rot, zf).reshape(BATCH, BLKS, 128)
        mB = jnp.where(partB, rot, zf).reshape(BATCH, BLKS, 128)
        o3 = (jax.lax.dot_general(
                  RA, mA, (((1,), (1,)), ((0,), (0,))),
                  preferred_element_type=jnp.float32,
                  precision=jax.lax.Precision.HIGHEST)
              + jax.lax.dot_general(
                  RB, mB, (((1,), (1,)), ((0,), (0,))),
                  preferred_element_type=jnp.float32,
                  precision=jax.lax.Precision.HIGHEST))
        return o3.reshape(BATCH, CW)

    clane = jax.lax.broadcasted_iota(jnp.int32, (BATCH, CW), 1)
    yy1 = compact(y1)
    xx1 = compact(x1)
    yy2 = compact(y2)
    xx2 = compact(x2)
    s_c = jnp.where(clane < PRE, compact(scores), jnp.float32(-1.0))
    ab = (jnp.float32(IOU_THR)
          * jnp.maximum(yy2 - yy1, 0.0) * jnp.maximum(xx2 - xx1, 0.0))

    T1 = 512
    key6 = _monotone_key(s_c)

    def count_ge6(thr):
        return jnp.sum((key6 >= thr).astype(jnp.int32), axis=1, keepdims=True)

    cur1 = jnp.where(count_ge6(zero) >= T1, zero, big)

    def bis1_body(k, cur):
        cand = cur | (jnp.int32(1) << (jnp.int32(30) - k))
        return jnp.where(count_ge6(cand) >= T1, cand, cur)

    thr2 = jax.lax.fori_loop(0, 31, bis1_body, cur1)
    gt2 = jnp.sum((key6 > thr2).astype(jnp.int32), axis=1, keepdims=True)
    need2 = T1 - gt2
    eq2 = key6 == thr2

    def idx1_body(k, cur):
        cand = cur | (jnp.int32(1) << (jnp.int32(12) - k))
        cnt = jnp.sum((eq2 & (clane < cand)).astype(jnp.int32), axis=1,
                      keepdims=True)
        return jnp.where(cnt < need2, cand, cur)

    idx2 = jax.lax.fori_loop(0, 13, idx1_body,
                             jnp.zeros((BATCH, 1), jnp.int32))
    valid1 = (key6 > thr2) | (eq2 & (clane <= idx2))

    BLK1 = CW // 128
    OR1 = T1 // 128
    vf1 = jnp.where(valid1, jnp.float32(1.0), zf)
    V1 = vf1.reshape(BATCH * BLK1, 128)
    rank1 = jax.lax.dot_general(
        V1, T128, (((1,), (0,)), ((), ())),
        preferred_element_type=jnp.float32,
        precision=jax.lax.Precision.HIGHEST)
    kidx1 = jax.lax.broadcasted_iota(
        jnp.int32, (BATCH * BLK1, 128), 1).astype(jnp.float32)
    lo1 = jnp.zeros((BATCH * BLK1, 128), jnp.int32)
    for bit in (64, 32, 16, 8, 4, 2, 1):
        c = lo1 + bit
        rc = jnp.take_along_axis(rank1, c, axis=1)
        lo1 = jnp.where(rc <= kidx1, c, lo1)
    cnt1 = rank1[:, 127:128] + V1[:, 127:128]
    C1 = jnp.sum(vf1.reshape(BATCH, BLK1, 128), axis=2)
    i48 = jax.lax.broadcasted_iota(jnp.int32, (BLK1, BLK1), 0)
    j48 = jax.lax.broadcasted_iota(jnp.int32, (BLK1, BLK1), 1)
    T48 = jnp.where(i48 < j48, jnp.float32(1.0), zf)
    O1 = jax.lax.dot_general(
        C1, T48, (((1,), (0,)), ((), ())),
        preferred_element_type=jnp.float32,
        precision=jax.lax.Precision.HIGHEST)
    O1i = O1.astype(jnp.int32)
    shift1 = jnp.broadcast_to((O1i & 127)[:, :, None],
                              (BATCH, BLK1, 128)).reshape(BATCH * BLK1, 128)
    lane1b = jax.lax.broadcasted_iota(jnp.int32, (BATCH * BLK1, 128), 1)
    idxrot1 = (lane1b - shift1 + 128) & 127
    real1 = idxrot1.astype(jnp.float32) < cnt1
    pA1 = real1 & (lane1b >= shift1)
    pB1 = real1 & (lane1b < shift1)
    m01 = O1i >> 7
    mm1 = jax.lax.broadcasted_iota(jnp.int32, (BATCH, BLK1, OR1), 2)
    RA1 = jnp.where(m01[:, :, None] == mm1, jnp.float32(1.0), zf)
    RB1 = jnp.where((m01 + 1)[:, :, None] == mm1, jnp.float32(1.0), zf)
    glo1 = jnp.take_along_axis(lo1, idxrot1, axis=1)

    def compact1(p):
        pp = p.reshape(BATCH * BLK1, 128)
        rot = jnp.take_along_axis(pp, glo1, axis=1)
        mA = jnp.where(pA1, rot, zf).reshape(BATCH, BLK1, 128)
        mB = jnp.where(pB1, rot, zf).reshape(BATCH, BLK1, 128)
        o3 = (jax.lax.dot_general(
                  RA1, mA, (((1,), (1,)), ((0,), (0,))),
                  preferred_element_type=jnp.float32,
                  precision=jax.lax.Precision.HIGHEST)
              + jax.lax.dot_general(
                  RB1, mB, (((1,), (1,)), ((0,), (0,))),
                  preferred_element_type=jnp.float32,
                  precision=jax.lax.Precision.HIGHEST))
        return o3.reshape(BATCH, T1)

    s1 = compact1(s_c)
    y1t = compact1(yy1)
    x1t = compact1(xx1)
    y2t = compact1(yy2)
    x2t = compact1(xx2)
    ab1 = (jnp.float32(IOU_THR)
           * jnp.maximum(y2t - y1t, 0.0) * jnp.maximum(x2t - x1t, 0.0))

    out_ref[...] = jnp.zeros((4, BATCH, OUT_PAD), jnp.float32)
    out_lane = jax.lax.broadcasted_iota(jnp.int32, (BATCH, OUT_PAD), 1)

    def make_pick(py1, px1, py2, px2, pab, plane_idx, W):
        def one_pick(s):
            m = jnp.max(s, axis=1, keepdims=True)
            anyv = m >= 0.0
            pick = (s == m) & anyv
            pos = jnp.min(jnp.where(pick, plane_idx, jnp.int32(W)), axis=1,
                          keepdims=True)
            onehot = plane_idx == pos
            oh_f = jnp.where(onehot, jnp.float32(1.0), zf)

            def sel(plane):
                return jnp.sum(plane * oh_f, axis=1, keepdims=True)

            by1 = sel(py1)
            bx1 = sel(px1)
            by2 = sel(py2)
            bx2 = sel(px2)
            inter = (jnp.maximum(jnp.minimum(by2, py2)
                                 - jnp.maximum(by1, py1), 0.0)
                     * jnp.maximum(jnp.minimum(bx2, px2)
                                   - jnp.maximum(bx1, px1), 0.0))
            area_a = (jnp.maximum(by2 - by1, 0.0)
                      * jnp.maximum(bx2 - bx1, 0.0))
            supp = (jnp.float32(1.0 + IOU_THR) * inter
                    > jnp.float32(IOU_THR) * area_a + pab)
            supp = supp | onehot
            s_next = jnp.where(anyv & supp, jnp.float32(-1.0), s)
            return s_next, (by1, bx1, by2, bx2), anyv
        return one_pick

    def write_out(o, i, box, anyv):
        wmask = (out_lane == i) & anyv
        return [jnp.where(wmask, jnp.clip(b, 0.0, 1.0), oo)
                for b, oo in zip(box, o)]

    lane_t1 = jax.lax.broadcasted_iota(jnp.int32, (BATCH, T1), 1)
    pick_t1 = make_pick(y1t, x1t, y2t, x2t, ab1, lane_t1, T1)

    def body1(i, carry):
        s, pk = carry
        o = [out_ref[0], out_ref[1], out_ref[2], out_ref[3]]
        for k in range(6):
            s, box, anyv = pick_t1(s)
            o = write_out(o, 6 * i + k, box, anyv)
            pk = pk + jnp.where(anyv, jnp.float32(1.0), zf)
        out_ref[...] = jnp.stack(o, axis=0)
        return s, pk

    _, pk = jax.lax.fori_loop(
        0, POST // 6, body1,
        (s1, jnp.zeros((BATCH, 1), jnp.float32)))

    need_fb = jnp.min(pk) < jnp.float32(POST)
    n_fb = jnp.where(need_fb, POST // 6, 0)
    pick_fw = make_pick(yy1, xx1, yy2, xx2, ab, clane, CW)

    def body2(j, s):
        o = [jnp.where(j == 0, zf, out_ref[k]) for k in range(4)]
        for k in range(6):
            s, box, anyv = pick_fw(s)
            o = write_out(o, 6 * j + k, box, anyv)
        out_ref[...] = jnp.stack(o, axis=0)
        return s

    jax.lax.fori_loop(0, n_fb, body2, s_c)


@jax.jit
def kernel(rpn_bbox_deltas, rpn_labels, anchors):
    deltas_t = jnp.transpose(rpn_bbox_deltas, (2, 0, 1))
    anchors_t = jnp.transpose(anchors, (1, 0))
    out = pl.pallas_call(
        _nms_kernel,
        out_shape=jax.ShapeDtypeStruct((4, BATCH, OUT_PAD), jnp.float32),
    )(rpn_labels, deltas_t, anchors_t)
    return jnp.transpose(out[:, :, :POST], (1, 2, 0))
